# Initial kernel scaffold; baseline (speedup 1.0000x reference)
import jax
import jax.numpy as jnp
from jax.experimental import pallas as pl


def kernel(x, edge_index, edge_attr, params):
    raise NotImplementedError("write your pallas kernel here")


# speedup vs baseline: 3.5416x; 3.5416x over previous
import functools

import jax
import jax.numpy as jnp
from jax import lax
from jax.experimental import pallas as pl
from jax.experimental.pallas import tpu as pltpu
from jax.experimental.pallas import tpu_sc as plsc

N = 10000
E = 160000
D = 256
HALF = 128
OUT_D = 3

NCORES = 2
NSUB = 16
EPT = E // NSUB
CB = 80
NCHUNK = EPT // CB

BR = 1000
GRID = N // BR


def _msg(x, w1, b1, w2, b2):
    h = jnp.maximum(jnp.dot(x, w1, preferred_element_type=jnp.float32) + b1, 0.0)
    return jnp.dot(h, w2, preferred_element_type=jnp.float32) + b2


def _enc_body(x_ref, we1, be1, we2, be2, we3, be3, g_ref, b_ref,
              w1a, b1e, w2, b2, x0_ref, ml_ref, mr_ref):
    h = jnp.maximum(jnp.dot(x_ref[...], we1[...], preferred_element_type=jnp.float32) + be1[...], 0.0)
    h = jnp.maximum(jnp.dot(h, we2[...], preferred_element_type=jnp.float32) + be2[...], 0.0)
    h = jnp.dot(h, we3[...], preferred_element_type=jnp.float32) + be3[...]
    mu = jnp.mean(h, axis=1, keepdims=True)
    var = jnp.mean((h - mu) * (h - mu), axis=1, keepdims=True)
    x0 = (h - mu) / jnp.sqrt(var + 1e-5) * g_ref[...] + b_ref[...]
    x0_ref[...] = x0
    mm = _msg(x0, w1a[...], b1e[...], w2[...], b2[...])
    ml_ref[...] = mm[:, :HALF]
    mr_ref[...] = mm[:, HALF:]


def _step_body(x_ref, sl_ref, sr_ref, r_ref, ws, bs,
               w1a, b1e, w2, b2, xt_ref, ml_ref, mr_ref):
    aggr = jnp.concatenate([sl_ref[...], sr_ref[...]], axis=1) * r_ref[...]
    xt = jnp.dot(x_ref[...], ws[...], preferred_element_type=jnp.float32) + bs[...] + aggr
    xt_ref[...] = xt
    mm = _msg(xt, w1a[...], b1e[...], w2[...], b2[...])
    ml_ref[...] = mm[:, :HALF]
    mr_ref[...] = mm[:, HALF:]


def _last_body(x_ref, sl_ref, sr_ref, r_ref, ws, bs,
               wd1, bd1, wd2, bd2, wd3, bd3, o_ref):
    aggr = jnp.concatenate([sl_ref[...], sr_ref[...]], axis=1) * r_ref[...]
    xt = jnp.dot(x_ref[...], ws[...], preferred_element_type=jnp.float32) + bs[...] + aggr
    h = jnp.maximum(jnp.dot(xt, wd1[...], preferred_element_type=jnp.float32) + bd1[...], 0.0)
    h = jnp.maximum(jnp.dot(h, wd2[...], preferred_element_type=jnp.float32) + bd2[...], 0.0)
    o_ref[...] = jnp.dot(h, wd3[...], preferred_element_type=jnp.float32) + bd3[...]


def _row_spec(width):
    return pl.BlockSpec((BR, width), lambda i: (i, 0))


def _full_spec(shape):
    return pl.BlockSpec(shape, lambda i: tuple(0 for _ in shape))


def _wspec(a):
    return _full_spec(a.shape)


def _f32(shape):
    return jax.ShapeDtypeStruct(shape, jnp.float32)


def _enc_call(x, weights):
    in_specs = [_row_spec(D)] + [_wspec(w) for w in weights]
    return pl.pallas_call(
        _enc_body,
        grid=(GRID,),
        in_specs=in_specs,
        out_specs=[_row_spec(D), _row_spec(HALF), _row_spec(HALF)],
        out_shape=[_f32((N, D)), _f32((N, HALF)), _f32((N, HALF))],
    )(x, *weights)


def _step_call(x, sl, sr, recip, weights):
    in_specs = [_row_spec(D), _row_spec(HALF), _row_spec(HALF), _row_spec(1)]
    in_specs += [_wspec(w) for w in weights]
    return pl.pallas_call(
        _step_body,
        grid=(GRID,),
        in_specs=in_specs,
        out_specs=[_row_spec(D), _row_spec(HALF), _row_spec(HALF)],
        out_shape=[_f32((N, D)), _f32((N, HALF)), _f32((N, HALF))],
    )(x, sl, sr, recip, *weights)


def _last_call(x, sl, sr, recip, weights):
    in_specs = [_row_spec(D), _row_spec(HALF), _row_spec(HALF), _row_spec(1)]
    in_specs += [_wspec(w) for w in weights]
    return pl.pallas_call(
        _last_body,
        grid=(GRID,),
        in_specs=in_specs,
        out_specs=[_row_spec(OUT_D)],
        out_shape=[_f32((N, OUT_D))],
    )(x, sl, sr, recip, *weights)[0]


_MESH = plsc.VectorSubcoreMesh(
    core_axis_name="c", subcore_axis_name="s",
    num_cores=NCORES, num_subcores=NSUB)

_NBLK = N // 8
_CNT_BLK = N // CB


def _half_pipeline(sid, m_hbm, src, dst, out_hbm, srcv, dstv, rows, zbuf,
                   acc, sem, cnt_parts):
    nz = jnp.where(sid < _NBLK % NSUB, _NBLK // NSUB + 1, _NBLK // NSUB)

    def zbody(i, _):
        blk = sid + i * NSUB
        pltpu.sync_copy(zbuf, acc.at[pl.ds(blk * 8, 8)])
        return ()
    lax.fori_loop(0, nz, zbody, (), unroll=False)

    if cnt_parts is not None:
        onesv, zc, acc_cnt, cnt_out, cwb = cnt_parts
        ncz = jnp.where(sid < _CNT_BLK % NSUB, _CNT_BLK // NSUB + 1,
                        _CNT_BLK // NSUB)

        def czbody(i, _):
            blk = sid + i * NSUB
            pltpu.sync_copy(zc, acc_cnt.at[pl.ds(blk * CB, CB)])
            return ()
        lax.fori_loop(0, ncz, czbody, (), unroll=False)

    plsc.subcore_barrier()

    ebase = sid * EPT

    def body(j, _):
        base = ebase + j * CB
        pltpu.sync_copy(src.at[pl.ds(base, CB)], srcv)
        pltpu.sync_copy(dst.at[pl.ds(base, CB)], dstv)
        pltpu.async_copy(m_hbm.at[srcv], rows, sem).wait()
        pltpu.sync_copy(rows, acc.at[dstv], add=True)
        if cnt_parts is not None:
            pltpu.sync_copy(cnt_parts[0], cnt_parts[2].at[dstv], add=True)
        return ()
    lax.fori_loop(0, NCHUNK, body, (), unroll=False)

    plsc.subcore_barrier()

    def wbody(i, _):
        blk = sid + i * NSUB
        pltpu.sync_copy(acc.at[pl.ds(blk * 8, 8)], out_hbm.at[pl.ds(blk * 8, 8)])
        return ()
    lax.fori_loop(0, nz, wbody, (), unroll=False)

    if cnt_parts is not None:
        onesv, zc, acc_cnt, cnt_out, cwb = cnt_parts

        def cwbody(i, _):
            blk = sid + i * NSUB
            pltpu.sync_copy(acc_cnt.at[pl.ds(blk * CB, CB)], cwb)
            pltpu.sync_copy(cwb, cnt_out.at[pl.ds(blk * CB, CB)])
            return ()
        lax.fori_loop(0, ncz, cwbody, (), unroll=False)


def _make_segsum(with_cnt):
    out_type = [_f32((N, HALF)), _f32((N, HALF))]
    scratch = [
        pltpu.VMEM((CB,), jnp.int32),
        pltpu.VMEM((CB,), jnp.int32),
        pltpu.VMEM((CB, HALF), jnp.float32),
        pltpu.VMEM((8, HALF), jnp.float32),
        pltpu.VMEM_SHARED((N, HALF), jnp.float32),
        pltpu.SemaphoreType.DMA,
    ]
    if with_cnt:
        out_type.append(_f32((N,)))
        scratch += [
            pltpu.VMEM((CB,), jnp.float32),
            pltpu.VMEM((CB,), jnp.float32),
            pltpu.VMEM_SHARED((N,), jnp.float32),
            pltpu.VMEM((CB,), jnp.float32),
        ]

    def body(ml, mr, src, dst, sl_out, sr_out, *rest):
        if with_cnt:
            cnt_out = rest[0]
            srcv, dstv, rows, zbuf, acc, sem, onesv, zc, acc_cnt, cwb = rest[1:]
        else:
            cnt_out = None
            srcv, dstv, rows, zbuf, acc, sem = rest

        cid = lax.axis_index("c")
        sid = lax.axis_index("s")

        zero16 = jnp.zeros((16,), jnp.float32)
        for i in range(8):
            for j in range(HALF // 16):
                zbuf[i, pl.ds(j * 16, 16)] = zero16
        if with_cnt:
            one16 = jnp.ones((16,), jnp.float32)
            for j in range(CB // 16):
                onesv[pl.ds(j * 16, 16)] = one16
                zc[pl.ds(j * 16, 16)] = zero16
            cnt_parts = (onesv, zc, acc_cnt, cnt_out, cwb)
        else:
            cnt_parts = None

        @pl.when(cid == 0)
        def _():
            _half_pipeline(sid, ml, src, dst, sl_out, srcv, dstv, rows,
                           zbuf, acc, sem, cnt_parts)

        @pl.when(cid == 1)
        def _():
            _half_pipeline(sid, mr, src, dst, sr_out, srcv, dstv, rows,
                           zbuf, acc, sem, None)

    return pl.kernel(body, out_type=out_type, mesh=_MESH,
                     scratch_types=scratch)


_segsum_cnt = _make_segsum(True)
_segsum = _make_segsum(False)


def _r1(b):
    return b.reshape(1, -1)


@jax.jit
def kernel(x, edge_index, edge_attr, params):
    src = edge_index[0]
    dst = edge_index[1]

    e_const = params["enc_edge"]["ln_b"][0]

    en = params["enc_node"]
    enc_w = []
    for l in en["mlp"]:
        enc_w += [l["w"], _r1(l["b"])]
    enc_w += [_r1(en["ln_g"]), _r1(en["ln_b"])]

    def msg_weights(t):
        lw = params["proc"][t]["edge_mlp"]
        w1 = lw[0]["w"]
        b1_eff = lw[0]["b"] + e_const * w1[D, :]
        return [w1[:D, :], _r1(b1_eff), lw[1]["w"], _r1(lw[1]["b"])]

    x0, ml, mr = _enc_call(x, enc_w + msg_weights(0))

    sl, sr, cnt = _segsum_cnt(ml, mr, src, dst)
    recip = (1.0 / jnp.maximum(cnt, 1.0)).reshape(N, 1)

    for t in range(3):
        ls = params["proc"][t]["lin_self"]
        w = [ls["w"], _r1(ls["b"])] + msg_weights(t + 1)
        x0, ml, mr = _step_call(x0, sl, sr, recip, w)
        sl, sr = _segsum(ml, mr, src, dst)

    ls = params["proc"][3]["lin_self"]
    w = [ls["w"], _r1(ls["b"])]
    for l in params["dec"]:
        w += [l["w"], _r1(l["b"])]
    return _last_call(x0, sl, sr, recip, w)


# speedup vs baseline: 9.4524x; 2.6690x over previous
import functools

import jax
import jax.numpy as jnp
from jax import lax
from jax.experimental import pallas as pl
from jax.experimental.pallas import tpu as pltpu
from jax.experimental.pallas import tpu_sc as plsc

N = 10000
E = 160000
D = 256
HALF = 128
OUT_D = 3

NCORES = 2
NSUB = 16
EPT = E // NSUB
CB = 125
NCHUNK = EPT // CB
CNT_CB = 80
ZROWS = 40
NZB = N // ZROWS
NGRP = 2
GCH = NCHUNK // NGRP

BR = 1000
GRID = N // BR


def _msg(x, w1, b1, w2, b2):
    h = jnp.maximum(jnp.dot(x, w1, preferred_element_type=jnp.float32) + b1, 0.0)
    return jnp.dot(h, w2, preferred_element_type=jnp.float32) + b2


def _enc_body(x_ref, we1, be1, we2, be2, we3, be3, g_ref, b_ref,
              w1a, b1e, w2, b2, x0_ref, ml_ref, mr_ref):
    h = jnp.maximum(jnp.dot(x_ref[...], we1[...], preferred_element_type=jnp.float32) + be1[...], 0.0)
    h = jnp.maximum(jnp.dot(h, we2[...], preferred_element_type=jnp.float32) + be2[...], 0.0)
    h = jnp.dot(h, we3[...], preferred_element_type=jnp.float32) + be3[...]
    mu = jnp.mean(h, axis=1, keepdims=True)
    var = jnp.mean((h - mu) * (h - mu), axis=1, keepdims=True)
    x0 = (h - mu) / jnp.sqrt(var + 1e-5) * g_ref[...] + b_ref[...]
    x0_ref[...] = x0
    mm = _msg(x0, w1a[...], b1e[...], w2[...], b2[...])
    ml_ref[...] = mm[:, :HALF]
    mr_ref[...] = mm[:, HALF:]


def _step_body(x_ref, sl_ref, sr_ref, r_ref, ws, bs,
               w1a, b1e, w2, b2, xt_ref, ml_ref, mr_ref):
    aggr = jnp.concatenate([sl_ref[...], sr_ref[...]], axis=1) * r_ref[...]
    xt = jnp.dot(x_ref[...], ws[...], preferred_element_type=jnp.float32) + bs[...] + aggr
    xt_ref[...] = xt
    mm = _msg(xt, w1a[...], b1e[...], w2[...], b2[...])
    ml_ref[...] = mm[:, :HALF]
    mr_ref[...] = mm[:, HALF:]


def _last_body(x_ref, sl_ref, sr_ref, r_ref, ws, bs,
               wd1, bd1, wd2, bd2, wd3, bd3, o_ref):
    aggr = jnp.concatenate([sl_ref[...], sr_ref[...]], axis=1) * r_ref[...]
    xt = jnp.dot(x_ref[...], ws[...], preferred_element_type=jnp.float32) + bs[...] + aggr
    h = jnp.maximum(jnp.dot(xt, wd1[...], preferred_element_type=jnp.float32) + bd1[...], 0.0)
    h = jnp.maximum(jnp.dot(h, wd2[...], preferred_element_type=jnp.float32) + bd2[...], 0.0)
    o_ref[...] = jnp.dot(h, wd3[...], preferred_element_type=jnp.float32) + bd3[...]


def _row_spec(width):
    return pl.BlockSpec((BR, width), lambda i: (i, 0))


def _full_spec(shape):
    return pl.BlockSpec(shape, lambda i: tuple(0 for _ in shape))


def _wspec(a):
    return _full_spec(a.shape)


def _f32(shape):
    return jax.ShapeDtypeStruct(shape, jnp.float32)


def _enc_call(x, weights):
    in_specs = [_row_spec(D)] + [_wspec(w) for w in weights]
    return pl.pallas_call(
        _enc_body,
        grid=(GRID,),
        in_specs=in_specs,
        out_specs=[_row_spec(D), _row_spec(HALF), _row_spec(HALF)],
        out_shape=[_f32((N, D)), _f32((N, HALF)), _f32((N, HALF))],
    )(x, *weights)


def _step_call(x, sl, sr, recip, weights):
    in_specs = [_row_spec(D), _row_spec(HALF), _row_spec(HALF), _row_spec(1)]
    in_specs += [_wspec(w) for w in weights]
    return pl.pallas_call(
        _step_body,
        grid=(GRID,),
        in_specs=in_specs,
        out_specs=[_row_spec(D), _row_spec(HALF), _row_spec(HALF)],
        out_shape=[_f32((N, D)), _f32((N, HALF)), _f32((N, HALF))],
    )(x, sl, sr, recip, *weights)


def _last_call(x, sl, sr, recip, weights):
    in_specs = [_row_spec(D), _row_spec(HALF), _row_spec(HALF), _row_spec(1)]
    in_specs += [_wspec(w) for w in weights]
    return pl.pallas_call(
        _last_body,
        grid=(GRID,),
        in_specs=in_specs,
        out_specs=[_row_spec(OUT_D)],
        out_shape=[_f32((N, OUT_D))],
    )(x, sl, sr, recip, *weights)[0]


_MESH = plsc.VectorSubcoreMesh(
    core_axis_name="c", subcore_axis_name="s",
    num_cores=NCORES, num_subcores=NSUB)

_CNT_BLK = N // CNT_CB


def _half_pipeline(sid, m_hbm, src3, dst3, out_hbm, srcv, dstv, rows0, rows1,
                   zbuf, acc, sem0, sem1, cnt_parts):
    nz = jnp.where(sid < NZB % NSUB, NZB // NSUB + 1, NZB // NSUB)

    def zbody(i, _):
        blk = sid + i * NSUB
        pltpu.sync_copy(zbuf, acc.at[pl.ds(blk * ZROWS, ZROWS)])
        return ()
    lax.fori_loop(0, nz, zbody, (), unroll=False)

    if cnt_parts is not None:
        onesv, zc, acc_cnt, cnt_out, cwb = cnt_parts
        ncz = jnp.where(sid < _CNT_BLK % NSUB, _CNT_BLK // NSUB + 1,
                        _CNT_BLK // NSUB)

        def czbody(i, _):
            blk = sid + i * NSUB
            pltpu.sync_copy(zc, acc_cnt.at[pl.ds(blk * CNT_CB, CNT_CB)])
            return ()
        lax.fori_loop(0, ncz, czbody, (), unroll=False)

    plsc.subcore_barrier()

    def fire(j, buf, sem):
        pltpu.async_copy(m_hbm.at[srcv.at[j]], buf, sem)

    def wait(buf, sem):
        pltpu.make_async_copy(m_hbm.at[srcv.at[0]], buf, sem).wait()

    def scat(j, buf):
        pltpu.sync_copy(buf, acc.at[dstv.at[j]], add=True)
        if cnt_parts is not None:
            pltpu.sync_copy(cnt_parts[0], cnt_parts[2].at[dstv.at[j]],
                            add=True)

    for g in range(NGRP):
        pltpu.sync_copy(src3.at[sid, pl.ds(g * GCH, GCH)], srcv)
        pltpu.sync_copy(dst3.at[sid, pl.ds(g * GCH, GCH)], dstv)

        fire(0, rows0, sem0)

        def body(i, _):
            j = 2 * i
            fire(j + 1, rows1, sem1)
            wait(rows0, sem0)
            scat(j, rows0)

            @pl.when(j + 2 < GCH)
            def _():
                fire(j + 2, rows0, sem0)
            wait(rows1, sem1)
            scat(j + 1, rows1)
            return ()
        lax.fori_loop(0, GCH // 2, body, (), unroll=False)

    plsc.subcore_barrier()

    def wbody(i, _):
        blk = sid + i * NSUB
        pltpu.sync_copy(acc.at[pl.ds(blk * ZROWS, ZROWS)],
                        out_hbm.at[pl.ds(blk * ZROWS, ZROWS)])
        return ()
    lax.fori_loop(0, nz, wbody, (), unroll=False)

    if cnt_parts is not None:
        onesv, zc, acc_cnt, cnt_out, cwb = cnt_parts

        def cwbody(i, _):
            blk = sid + i * NSUB
            pltpu.sync_copy(acc_cnt.at[pl.ds(blk * CNT_CB, CNT_CB)], cwb)
            pltpu.sync_copy(cwb, cnt_out.at[pl.ds(blk * CNT_CB, CNT_CB)])
            return ()
        lax.fori_loop(0, ncz, cwbody, (), unroll=False)


def _fill(ref, length, value):
    v = jnp.full((16,), value, jnp.float32)
    for j in range(0, length - 15, 16):
        ref[pl.ds(j, 16)] = v
    if length % 16:
        ref[pl.ds(length - 16, 16)] = v


def _make_segsum(with_cnt):
    out_type = [_f32((N, HALF)), _f32((N, HALF))]
    scratch = [
        pltpu.VMEM((GCH, CB), jnp.int32),
        pltpu.VMEM((GCH, CB), jnp.int32),
        pltpu.VMEM((CB, HALF), jnp.float32),
        pltpu.VMEM((CB, HALF), jnp.float32),
        pltpu.VMEM((ZROWS, HALF), jnp.float32),
        pltpu.VMEM_SHARED((N, HALF), jnp.float32),
        pltpu.SemaphoreType.DMA,
        pltpu.SemaphoreType.DMA,
    ]
    if with_cnt:
        out_type.append(_f32((N,)))
        scratch += [
            pltpu.VMEM((CB,), jnp.float32),
            pltpu.VMEM((CNT_CB,), jnp.float32),
            pltpu.VMEM_SHARED((N,), jnp.float32),
            pltpu.VMEM((CNT_CB,), jnp.float32),
        ]

    def body(ml, mr, src3, dst3, sl_out, sr_out, *rest):
        if with_cnt:
            cnt_out = rest[0]
            (srcv, dstv, rows0, rows1, zbuf, acc, sem0, sem1,
             onesv, zc, acc_cnt, cwb) = rest[1:]
        else:
            cnt_out = None
            srcv, dstv, rows0, rows1, zbuf, acc, sem0, sem1 = rest

        cid = lax.axis_index("c")
        sid = lax.axis_index("s")

        zero16 = jnp.zeros((16,), jnp.float32)

        def zfill(i, _):
            for j in range(HALF // 16):
                zbuf[i, pl.ds(j * 16, 16)] = zero16
            return ()
        lax.fori_loop(0, ZROWS, zfill, (), unroll=False)
        if with_cnt:
            _fill(onesv, CB, 1.0)
            _fill(zc, CNT_CB, 0.0)
            cnt_parts = (onesv, zc, acc_cnt, cnt_out, cwb)
        else:
            cnt_parts = None

        @pl.when(cid == 0)
        def _():
            _half_pipeline(sid, ml, src3, dst3, sl_out, srcv, dstv,
                           rows0, rows1, zbuf, acc, sem0, sem1, cnt_parts)

        @pl.when(cid == 1)
        def _():
            _half_pipeline(sid, mr, src3, dst3, sr_out, srcv, dstv,
                           rows0, rows1, zbuf, acc, sem0, sem1, None)

    return pl.kernel(body, out_type=out_type, mesh=_MESH,
                     scratch_types=scratch)


_segsum_cnt = _make_segsum(True)
_segsum = _make_segsum(False)


def _r1(b):
    return b.reshape(1, -1)


@jax.jit
def kernel(x, edge_index, edge_attr, params):
    src = edge_index[0].reshape(NSUB, NCHUNK, CB)
    dst = edge_index[1].reshape(NSUB, NCHUNK, CB)

    e_const = params["enc_edge"]["ln_b"][0]

    en = params["enc_node"]
    enc_w = []
    for l in en["mlp"]:
        enc_w += [l["w"], _r1(l["b"])]
    enc_w += [_r1(en["ln_g"]), _r1(en["ln_b"])]

    def msg_weights(t):
        lw = params["proc"][t]["edge_mlp"]
        w1 = lw[0]["w"]
        b1_eff = lw[0]["b"] + e_const * w1[D, :]
        return [w1[:D, :], _r1(b1_eff), lw[1]["w"], _r1(lw[1]["b"])]

    x0, ml, mr = _enc_call(x, enc_w + msg_weights(0))

    sl, sr, cnt = _segsum_cnt(ml, mr, src, dst)
    recip = (1.0 / jnp.maximum(cnt, 1.0)).reshape(N, 1)

    for t in range(3):
        ls = params["proc"][t]["lin_self"]
        w = [ls["w"], _r1(ls["b"])] + msg_weights(t + 1)
        x0, ml, mr = _step_call(x0, sl, sr, recip, w)
        sl, sr = _segsum(ml, mr, src, dst)

    ls = params["proc"][3]["lin_self"]
    w = [ls["w"], _r1(ls["b"])]
    for l in params["dec"]:
        w += [l["w"], _r1(l["b"])]
    return _last_call(x0, sl, sr, recip, w)
